# baseline (speedup 1.0000x reference)
import functools

import jax
import jax.numpy as jnp
from jax import lax
from jax.experimental import pallas as pl
from jax.experimental.pallas import tpu as pltpu
from jax.experimental.pallas import tpu_sc as plsc

DEPTH = 200
PER_W = 8
N_WORKERS = DEPTH // PER_W
LANES = 16
DIM = 16
SLAB = 128


def _body(inputs_hbm, label_hbm, emb1t_hbm, emb2t_hbm,
          out_sig_hbm, out_tgt_hbm,
          idx_v, xidx_v, slabs_v, pslab_v, label_v, sig_v, tgt_v, sem):
    wid = lax.axis_index("s") * 2 + lax.axis_index("c")

    @pl.when(wid < N_WORKERS)
    def _():
        base = wid * PER_W
        st1 = pltpu.async_copy(inputs_hbm.at[1, pl.ds(base, PER_W)],
                               idx_v.at[pl.ds(0, PER_W)], sem)
        st2 = pltpu.async_copy(inputs_hbm.at[0, pl.ds(0, PER_W)],
                               xidx_v.at[pl.ds(0, PER_W)], sem)
        st3 = pltpu.async_copy(label_hbm.at[0, pl.ds(base, PER_W)],
                               label_v.at[pl.ds(0, PER_W)], sem)
        st1.wait()
        st2.wait()
        st3.wait()

        lanes = lax.iota(jnp.int32, LANES)
        active = lanes < PER_W
        idxr = jnp.where(active, idx_v[...], 0)
        x0 = xidx_v[...][0]
        xstart = (x0 // SLAB) * SLAB
        copies = [pltpu.async_copy(
            emb1t_hbm.at[:, pl.ds(xstart, SLAB)], pslab_v, sem)]
        for i in range(PER_W):
            vi = idxr[i]
            vstart = (vi // SLAB) * SLAB
            copies.append(pltpu.async_copy(
                emb2t_hbm.at[:, pl.ds(vstart, SLAB)],
                slabs_v.at[pl.ds(i * DIM, DIM), :], sem))
        for cp in copies:
            cp.wait()

        subv = jnp.bitwise_and(idxr, SLAB - 1)
        xsub = jnp.bitwise_and(x0, SLAB - 1)
        pvec = plsc.load_gather(pslab_v, [lanes, jnp.full((LANES,), xsub)])
        slot = jnp.where(active, lanes, 0)
        acc = jnp.zeros((LANES,), jnp.float32)
        for f in range(DIM):
            vals = plsc.load_gather(slabs_v, [slot * DIM + f, subv])
            acc = acc + vals * pvec[f]

        sig = 1.0 / (1.0 + jnp.exp(-acc))
        mask_i = (sig >= 0.5).astype(jnp.int32)
        lbl = label_v[...]
        tgt = (mask_i == lbl).astype(jnp.int32)

        sig_v[...] = sig
        tgt_v[...] = tgt
        pltpu.sync_copy(sig_v.at[pl.ds(0, PER_W)],
                        out_sig_hbm.at[0, pl.ds(base, PER_W)])
        pltpu.sync_copy(tgt_v.at[pl.ds(0, PER_W)],
                        out_tgt_hbm.at[0, pl.ds(base, PER_W)])


@jax.jit
def kernel(inputs, label, embedding_1, embedding_2):
    mesh = plsc.VectorSubcoreMesh(core_axis_name="c", subcore_axis_name="s")
    run = functools.partial(
        pl.kernel,
        out_type=[
            jax.ShapeDtypeStruct((1, DEPTH), jnp.float32),
            jax.ShapeDtypeStruct((1, DEPTH), jnp.int32),
        ],
        mesh=mesh,
        compiler_params=pltpu.CompilerParams(
            needs_layout_passes=False,
            disable_bounds_checks=True,
            disable_semaphore_checks=True),
        scratch_types=[
            pltpu.VMEM((LANES,), jnp.int32),
            pltpu.VMEM((LANES,), jnp.int32),
            pltpu.VMEM((PER_W * DIM, SLAB), jnp.float32),
            pltpu.VMEM((DIM, SLAB), jnp.float32),
            pltpu.VMEM((LANES,), jnp.int32),
            pltpu.VMEM((LANES,), jnp.float32),
            pltpu.VMEM((LANES,), jnp.int32),
            pltpu.SemaphoreType.DMA,
        ],
    )(_body)
    sig, tgt = run(inputs.astype(jnp.int32), label.astype(jnp.int32),
                   embedding_1.T, embedding_2.T)
    return (sig, tgt.astype(label.dtype))
